# Initial kernel scaffold; baseline (speedup 1.0000x reference)
import jax
import jax.numpy as jnp
from jax.experimental import pallas as pl


def kernel(x, e_type, e_index):
    raise NotImplementedError("write your pallas kernel here")


# speedup vs baseline: 28.0115x; 28.0115x over previous
import functools

import jax
import jax.numpy as jnp
from jax import lax
from jax.experimental import pallas as pl
from jax.experimental.pallas import tpu as pltpu
from jax.experimental.pallas import tpu_sc as plsc

N_NODES = 100000
N_EDGES = 6400000
DIST_MIN = 1.2
DIST_MAX = 1.9
EPS1 = 0.1
EPS2 = 0.1

NC = 2
NS = 16
L = 16
NW = NC * NS
PER_W = N_EDGES // NW
CHUNK = 2000
NCHUNKS = PER_W // CHUNK
GROUPS = CHUNK // L

_mesh = plsc.VectorSubcoreMesh(core_axis_name="c", subcore_axis_name="s")


@functools.partial(
    pl.kernel,
    mesh=_mesh,
    out_type=jax.ShapeDtypeStruct((NW, L), jnp.float32),
    scratch_types=[
        pltpu.VMEM((CHUNK,), jnp.int32),
        pltpu.VMEM((CHUNK,), jnp.int32),
        pltpu.VMEM((CHUNK,), jnp.int32),
        pltpu.VMEM((CHUNK,), jnp.float32),
        pltpu.VMEM((CHUNK,), jnp.float32),
        pltpu.VMEM((CHUNK,), jnp.float32),
        pltpu.VMEM((CHUNK,), jnp.float32),
        pltpu.VMEM((CHUNK,), jnp.float32),
        pltpu.VMEM((CHUNK,), jnp.float32),
        pltpu.VMEM((L,), jnp.float32),
        pltpu.SemaphoreType.DMA,
    ],
)
def _drift_kernel(x0_hbm, x1_hbm, x2_hbm, ei_hbm, et_hbm, out_hbm,
                  sidx_v, didx_v, et_v,
                  sx_v, sy_v, sz_v, tx_v, ty_v, tz_v,
                  acc_v, sem):
    cid = lax.axis_index("c")
    sid = lax.axis_index("s")
    wid = sid * NC + cid
    base_w = wid * PER_W

    def chunk_body(ci, acc):
        base = base_w + ci * CHUNK
        pltpu.sync_copy(ei_hbm.at[pl.ds(base, CHUNK)], sidx_v)
        pltpu.sync_copy(ei_hbm.at[pl.ds(N_EDGES + base, CHUNK)], didx_v)
        pltpu.sync_copy(et_hbm.at[pl.ds(base, CHUNK)], et_v)
        cps = [
            pltpu.async_copy(x0_hbm.at[sidx_v], sx_v, sem),
            pltpu.async_copy(x1_hbm.at[sidx_v], sy_v, sem),
            pltpu.async_copy(x2_hbm.at[sidx_v], sz_v, sem),
            pltpu.async_copy(x0_hbm.at[didx_v], tx_v, sem),
            pltpu.async_copy(x1_hbm.at[didx_v], ty_v, sem),
            pltpu.async_copy(x2_hbm.at[didx_v], tz_v, sem),
        ]
        for cp in cps:
            cp.wait()

        def group_body(gi, acc_in):
            o = pl.ds(gi * L, L)
            dx = sx_v[o] - tx_v[o]
            dy = sy_v[o] - ty_v[o]
            dz = sz_v[o] - tz_v[o]
            et = et_v[o]
            d2 = dx * dx + dy * dy + dz * dz

            bits = lax.bitcast_convert_type(d2, jnp.int32)
            seed = jnp.int32(0x5F3759DF) - lax.shift_right_arithmetic(bits, 1)
            y = lax.bitcast_convert_type(seed, jnp.float32)
            h = 0.5 * d2
            y = y * (1.5 - h * y * y)
            y = y * (1.5 - h * y * y)
            y = y * (1.5 - h * y * y)
            d = jnp.where(d2 > 0.0, d2 * y, 0.0)

            drift = EPS1 * jnp.maximum(d - DIST_MAX, 0.0) + \
                EPS2 * jnp.maximum(DIST_MIN - d, 0.0)
            drift = jnp.where(et == 0, 0.0, drift)
            return acc_in + drift

        return lax.fori_loop(0, GROUPS, group_body, acc)

    acc = lax.fori_loop(0, NCHUNKS, chunk_body, jnp.zeros((L,), jnp.float32))
    acc_v[...] = acc
    pltpu.sync_copy(acc_v, out_hbm.at[wid])


def kernel(x, e_type, e_index):
    x0 = x[:, 0]
    x1 = x[:, 1]
    x2 = x[:, 2]
    parts = _drift_kernel(x0, x1, x2,
                          e_index.astype(jnp.int32).reshape(-1),
                          e_type.astype(jnp.int32))
    return parts.sum()


# speedup vs baseline: 73.6652x; 2.6298x over previous
import functools

import jax
import jax.numpy as jnp
from jax import lax
from jax.experimental import pallas as pl
from jax.experimental.pallas import tpu as pltpu
from jax.experimental.pallas import tpu_sc as plsc

N_NODES = 100000
N_EDGES = 6400000
DIST_MIN = 1.2
DIST_MAX = 1.9
EPS1 = 0.1
EPS2 = 0.1

NC = 2
NS = 16
L = 16
NW = NC * NS
PER_W = N_EDGES // NW
CHUNK = 2000
NCHUNKS = PER_W // CHUNK
GROUPS = CHUNK // L

_mesh = plsc.VectorSubcoreMesh(core_axis_name="c", subcore_axis_name="s")


@functools.partial(
    pl.kernel,
    mesh=_mesh,
    out_type=jax.ShapeDtypeStruct((NW, L), jnp.float32),
    scratch_types=[
        pltpu.VMEM((CHUNK,), jnp.int32),
        pltpu.VMEM((CHUNK,), jnp.int32),
        pltpu.VMEM((CHUNK,), jnp.int32),
        pltpu.VMEM((CHUNK,), jnp.float32),
        pltpu.VMEM((CHUNK,), jnp.float32),
        pltpu.VMEM((CHUNK,), jnp.float32),
        pltpu.VMEM((CHUNK,), jnp.float32),
        pltpu.VMEM((CHUNK,), jnp.float32),
        pltpu.VMEM((CHUNK,), jnp.float32),
        pltpu.VMEM((L,), jnp.float32),
        pltpu.VMEM_SHARED((N_NODES,), jnp.float32),
        pltpu.VMEM_SHARED((N_NODES,), jnp.float32),
        pltpu.VMEM_SHARED((N_NODES,), jnp.float32),
        pltpu.SemaphoreType.DMA,
    ],
)
def _drift_kernel(x0_hbm, x1_hbm, x2_hbm, ei_hbm, et_hbm, out_hbm,
                  sidx_v, didx_v, et_v,
                  sx_v, sy_v, sz_v, tx_v, ty_v, tz_v,
                  acc_v, x0_sh, x1_sh, x2_sh, sem):
    cid = lax.axis_index("c")
    sid = lax.axis_index("s")
    wid = sid * NC + cid
    base_w = wid * PER_W

    @pl.when(sid == 0)
    def _stage():
        pltpu.sync_copy(x0_hbm, x0_sh)
        pltpu.sync_copy(x1_hbm, x1_sh)
        pltpu.sync_copy(x2_hbm, x2_sh)

    plsc.subcore_barrier()

    def chunk_body(ci, acc):
        base = base_w + ci * CHUNK
        pltpu.sync_copy(ei_hbm.at[pl.ds(base, CHUNK)], sidx_v)
        pltpu.sync_copy(ei_hbm.at[pl.ds(N_EDGES + base, CHUNK)], didx_v)
        pltpu.sync_copy(et_hbm.at[pl.ds(base, CHUNK)], et_v)
        cps = [
            pltpu.async_copy(x0_sh.at[sidx_v], sx_v, sem),
            pltpu.async_copy(x1_sh.at[sidx_v], sy_v, sem),
            pltpu.async_copy(x2_sh.at[sidx_v], sz_v, sem),
            pltpu.async_copy(x0_sh.at[didx_v], tx_v, sem),
            pltpu.async_copy(x1_sh.at[didx_v], ty_v, sem),
            pltpu.async_copy(x2_sh.at[didx_v], tz_v, sem),
        ]
        for cp in cps:
            cp.wait()

        def group_body(gi, acc_in):
            o = pl.ds(gi * L, L)
            dx = sx_v[o] - tx_v[o]
            dy = sy_v[o] - ty_v[o]
            dz = sz_v[o] - tz_v[o]
            et = et_v[o]
            d2 = dx * dx + dy * dy + dz * dz

            bits = lax.bitcast_convert_type(d2, jnp.int32)
            seed = jnp.int32(0x5F3759DF) - lax.shift_right_arithmetic(bits, 1)
            y = lax.bitcast_convert_type(seed, jnp.float32)
            h = 0.5 * d2
            y = y * (1.5 - h * y * y)
            y = y * (1.5 - h * y * y)
            y = y * (1.5 - h * y * y)
            d = jnp.where(d2 > 0.0, d2 * y, 0.0)

            drift = EPS1 * jnp.maximum(d - DIST_MAX, 0.0) + \
                EPS2 * jnp.maximum(DIST_MIN - d, 0.0)
            drift = jnp.where(et == 0, 0.0, drift)
            return acc_in + drift

        return lax.fori_loop(0, GROUPS, group_body, acc)

    acc = lax.fori_loop(0, NCHUNKS, chunk_body, jnp.zeros((L,), jnp.float32))
    acc_v[...] = acc
    pltpu.sync_copy(acc_v, out_hbm.at[wid])


def kernel(x, e_type, e_index):
    x0 = x[:, 0]
    x1 = x[:, 1]
    x2 = x[:, 2]
    parts = _drift_kernel(x0, x1, x2,
                          e_index.astype(jnp.int32).reshape(-1),
                          e_type.astype(jnp.int32))
    return parts.sum()


# speedup vs baseline: 120.6346x; 1.6376x over previous
import functools

import jax
import jax.numpy as jnp
from jax import lax
from jax.experimental import pallas as pl
from jax.experimental.pallas import tpu as pltpu
from jax.experimental.pallas import tpu_sc as plsc

N_NODES = 100000
N_EDGES = 6400000
DIST_MIN = 1.2
DIST_MAX = 1.9
EPS1 = 0.1
EPS2 = 0.1

NC = 2
NS = 16
L = 16
NW = NC * NS
PER_W = N_EDGES // NW
CHUNK = 2000
NCHUNKS = PER_W // CHUNK
GROUPS = CHUNK // L

_mesh = plsc.VectorSubcoreMesh(core_axis_name="c", subcore_axis_name="s")

_IDX = lambda: pltpu.VMEM((CHUNK,), jnp.int32)
_DAT = lambda: pltpu.VMEM((CHUNK,), jnp.float32)


@functools.partial(
    pl.kernel,
    mesh=_mesh,
    out_type=jax.ShapeDtypeStruct((NW, L), jnp.float32),
    scratch_types=[
        _IDX(), _IDX(),
        _IDX(), _IDX(),
        _IDX(), _IDX(),
        _DAT(), _DAT(), _DAT(),
        _DAT(), _DAT(), _DAT(),
        _DAT(), _DAT(), _DAT(),
        _DAT(), _DAT(), _DAT(),
        pltpu.VMEM((L,), jnp.float32),
        pltpu.VMEM_SHARED((N_NODES,), jnp.float32),
        pltpu.VMEM_SHARED((N_NODES,), jnp.float32),
        pltpu.VMEM_SHARED((N_NODES,), jnp.float32),
        pltpu.SemaphoreType.DMA,
        pltpu.SemaphoreType.DMA,
    ],
)
def _drift_kernel(x0_hbm, x1_hbm, x2_hbm, ei_hbm, et_hbm, out_hbm,
                  si0, si1, di0, di1, et0, et1,
                  sx0, sy0, sz0, tx0, ty0, tz0,
                  sx1, sy1, sz1, tx1, ty1, tz1,
                  acc_v, x0_sh, x1_sh, x2_sh, sem_lin, sem_gat):
    cid = lax.axis_index("c")
    sid = lax.axis_index("s")
    wid = sid * NC + cid
    base_w = wid * PER_W

    idx_set = ((si0, di0), (si1, di1))
    dat_set = ((et0, sx0, sy0, sz0, tx0, ty0, tz0),
               (et1, sx1, sy1, sz1, tx1, ty1, tz1))

    @pl.when(sid == 0)
    def _stage():
        pltpu.sync_copy(x0_hbm, x0_sh)
        pltpu.sync_copy(x1_hbm, x1_sh)
        pltpu.sync_copy(x2_hbm, x2_sh)

    plsc.subcore_barrier()

    def issue_lin(ci, p):
        base = base_w + ci * CHUNK
        si, di = idx_set[p]
        c1 = pltpu.make_async_copy(ei_hbm.at[pl.ds(base, CHUNK)], si, sem_lin)
        c2 = pltpu.make_async_copy(ei_hbm.at[pl.ds(N_EDGES + base, CHUNK)],
                                   di, sem_lin)
        c1.start()
        c2.start()
        return (c1, c2)

    def issue_gat(ci, p):
        base = base_w + ci * CHUNK
        si, di = idx_set[p]
        et, sx, sy, sz, tx, ty, tz = dat_set[p]
        cps = [
            pltpu.make_async_copy(et_hbm.at[pl.ds(base, CHUNK)], et, sem_lin),
            pltpu.make_async_copy(x0_sh.at[si], sx, sem_gat),
            pltpu.make_async_copy(x1_sh.at[si], sy, sem_gat),
            pltpu.make_async_copy(x2_sh.at[si], sz, sem_gat),
            pltpu.make_async_copy(x0_sh.at[di], tx, sem_gat),
            pltpu.make_async_copy(x1_sh.at[di], ty, sem_gat),
            pltpu.make_async_copy(x2_sh.at[di], tz, sem_gat),
        ]
        for cp in cps:
            cp.start()
        return cps

    def wait_all(cps):
        for cp in cps:
            cp.wait()

    def compute(p, acc):
        et, sx, sy, sz, tx, ty, tz = dat_set[p]

        def group_body(gi, acc_in):
            o = pl.ds(gi * L, L)
            dx = sx[o] - tx[o]
            dy = sy[o] - ty[o]
            dz = sz[o] - tz[o]
            etv = et[o]
            d2 = dx * dx + dy * dy + dz * dz

            bits = lax.bitcast_convert_type(d2, jnp.int32)
            seed = jnp.int32(0x5F3759DF) - lax.shift_right_arithmetic(bits, 1)
            y = lax.bitcast_convert_type(seed, jnp.float32)
            h = 0.5 * d2
            y = y * (1.5 - h * y * y)
            y = y * (1.5 - h * y * y)
            d = jnp.where(d2 > 0.0, d2 * y, 0.0)

            drift = EPS1 * jnp.maximum(d - DIST_MAX, 0.0) + \
                EPS2 * jnp.maximum(DIST_MIN - d, 0.0)
            drift = jnp.where(etv == 0, 0.0, drift)
            return acc_in + drift

        return lax.fori_loop(0, GROUPS, group_body, acc, unroll=4)

    wait_all(issue_lin(0, 0))
    cps_l1 = issue_lin(1, 1)
    cps_g0 = issue_gat(0, 0)
    wait_all(cps_l1)
    wait_all(cps_g0)

    def pair_body(i, acc):
        ci = 2 * i
        cps_l = issue_lin(ci + 2, 0)
        cps_g = issue_gat(ci + 1, 1)
        acc = compute(0, acc)
        wait_all(cps_g)
        wait_all(cps_l)
        cps_l = issue_lin(ci + 3, 1)
        cps_g = issue_gat(ci + 2, 0)
        acc = compute(1, acc)
        wait_all(cps_g)
        wait_all(cps_l)
        return acc

    acc = lax.fori_loop(0, NCHUNKS // 2 - 1, pair_body,
                        jnp.zeros((L,), jnp.float32))
    cps_g = issue_gat(NCHUNKS - 1, 1)
    acc = compute(0, acc)
    wait_all(cps_g)
    acc = compute(1, acc)
    acc_v[...] = acc
    pltpu.sync_copy(acc_v, out_hbm.at[wid])


def kernel(x, e_type, e_index):
    x0 = x[:, 0]
    x1 = x[:, 1]
    x2 = x[:, 2]
    parts = _drift_kernel(x0, x1, x2,
                          e_index.astype(jnp.int32).reshape(-1),
                          e_type.astype(jnp.int32))
    return parts.sum()


# speedup vs baseline: 122.3068x; 1.0139x over previous
import functools

import jax
import jax.numpy as jnp
from jax import lax
from jax.experimental import pallas as pl
from jax.experimental.pallas import tpu as pltpu
from jax.experimental.pallas import tpu_sc as plsc

N_NODES = 100000
N_EDGES = 6400000
DIST_MIN = 1.2
DIST_MAX = 1.9
EPS = 0.1

NC = 2
NS = 16
L = 16
NW = NC * NS
PER_W = N_EDGES // NW
CHUNK = 4000
NCHUNKS = PER_W // CHUNK
GROUPS = CHUNK // L

_mesh = plsc.VectorSubcoreMesh(core_axis_name="c", subcore_axis_name="s")

_IDX = lambda: pltpu.VMEM((CHUNK,), jnp.int32)
_DAT = lambda: pltpu.VMEM((CHUNK,), jnp.float32)


@functools.partial(
    pl.kernel,
    mesh=_mesh,
    out_type=jax.ShapeDtypeStruct((NW, L), jnp.float32),
    scratch_types=[
        _IDX(), _IDX(),
        _IDX(), _IDX(),
        _IDX(), _IDX(),
        _DAT(), _DAT(), _DAT(),
        _DAT(), _DAT(), _DAT(),
        _DAT(), _DAT(), _DAT(),
        _DAT(), _DAT(), _DAT(),
        pltpu.VMEM((L,), jnp.float32),
        pltpu.VMEM_SHARED((N_NODES,), jnp.float32),
        pltpu.VMEM_SHARED((N_NODES,), jnp.float32),
        pltpu.VMEM_SHARED((N_NODES,), jnp.float32),
        pltpu.SemaphoreType.DMA,
        pltpu.SemaphoreType.DMA,
    ],
)
def _drift_kernel(x0_hbm, x1_hbm, x2_hbm, ei_hbm, et_hbm, out_hbm,
                  si0, si1, di0, di1, et0, et1,
                  sx0, sy0, sz0, tx0, ty0, tz0,
                  sx1, sy1, sz1, tx1, ty1, tz1,
                  acc_v, x0_sh, x1_sh, x2_sh, sem_lin, sem_gat):
    cid = lax.axis_index("c")
    sid = lax.axis_index("s")
    wid = sid * NC + cid
    base_w = wid * PER_W

    idx_set = ((si0, di0), (si1, di1))
    dat_set = ((et0, sx0, sy0, sz0, tx0, ty0, tz0),
               (et1, sx1, sy1, sz1, tx1, ty1, tz1))

    @pl.when(sid == 0)
    def _stage():
        pltpu.sync_copy(x0_hbm, x0_sh)
        pltpu.sync_copy(x1_hbm, x1_sh)
        pltpu.sync_copy(x2_hbm, x2_sh)

    plsc.subcore_barrier()

    def issue_lin(ci, p):
        base = base_w + ci * CHUNK
        si, di = idx_set[p]
        cps = [
            pltpu.make_async_copy(ei_hbm.at[pl.ds(base, CHUNK)], si, sem_lin),
            pltpu.make_async_copy(ei_hbm.at[pl.ds(N_EDGES + base, CHUNK)],
                                  di, sem_lin),
        ]
        for cp in cps:
            cp.start()
        return cps

    def issue_gat(ci, p):
        base = base_w + ci * CHUNK
        si, di = idx_set[p]
        et, sx, sy, sz, tx, ty, tz = dat_set[p]
        cps = [
            pltpu.make_async_copy(et_hbm.at[pl.ds(base, CHUNK)], et, sem_lin),
            pltpu.make_async_copy(x0_sh.at[si], sx, sem_gat),
            pltpu.make_async_copy(x1_sh.at[si], sy, sem_gat),
            pltpu.make_async_copy(x2_sh.at[si], sz, sem_gat),
            pltpu.make_async_copy(x0_sh.at[di], tx, sem_gat),
            pltpu.make_async_copy(x1_sh.at[di], ty, sem_gat),
            pltpu.make_async_copy(x2_sh.at[di], tz, sem_gat),
        ]
        for cp in cps:
            cp.start()
        return cps

    def wait_all(cps):
        for cp in cps:
            cp.wait()

    def compute(p, acc):
        et, sx, sy, sz, tx, ty, tz = dat_set[p]

        def group_body(gi, acc_in):
            o = pl.ds(gi * L, L)
            dx = sx[o] - tx[o]
            dy = sy[o] - ty[o]
            dz = sz[o] - tz[o]
            etv = et[o]
            d2 = jnp.maximum(dx * dx + dy * dy + dz * dz, 1e-30)

            bits = lax.bitcast_convert_type(d2, jnp.int32)
            seed = jnp.int32(0x5F3759DF) - lax.shift_right_arithmetic(bits, 1)
            y = lax.bitcast_convert_type(seed, jnp.float32)
            h = 0.5 * d2
            y = y * (1.5 - h * y * y)
            y = y * (1.5 - h * y * y)
            d = d2 * y

            drift = jnp.maximum(d - DIST_MAX,
                                jnp.maximum(DIST_MIN - d, 0.0))
            drift = jnp.where(etv == 0, 0.0, drift)
            return acc_in + drift

        return lax.fori_loop(0, GROUPS, group_body, acc, unroll=4)

    wait_all(issue_lin(0, 0))
    cps_l1 = issue_lin(1, 1)
    cps_g0 = issue_gat(0, 0)
    wait_all(cps_l1)
    wait_all(cps_g0)

    def pair_body(i, acc):
        ci = 2 * i
        cps_l = issue_lin(ci + 2, 0)
        cps_g = issue_gat(ci + 1, 1)
        acc = compute(0, acc)
        wait_all(cps_g)
        wait_all(cps_l)
        cps_l = issue_lin(ci + 3, 1)
        cps_g = issue_gat(ci + 2, 0)
        acc = compute(1, acc)
        wait_all(cps_g)
        wait_all(cps_l)
        return acc

    acc = lax.fori_loop(0, NCHUNKS // 2 - 1, pair_body,
                        jnp.zeros((L,), jnp.float32))
    cps_g = issue_gat(NCHUNKS - 1, 1)
    acc = compute(0, acc)
    wait_all(cps_g)
    acc = compute(1, acc)

    acc_v[...] = acc * EPS
    pltpu.sync_copy(acc_v, out_hbm.at[wid])


def kernel(x, e_type, e_index):
    x0 = x[:, 0]
    x1 = x[:, 1]
    x2 = x[:, 2]
    parts = _drift_kernel(x0, x1, x2,
                          e_index.astype(jnp.int32).reshape(-1),
                          e_type.astype(jnp.int32))
    return parts.sum()


# speedup vs baseline: 136.3408x; 1.1147x over previous
import functools

import jax
import jax.numpy as jnp
from jax import lax
from jax.experimental import pallas as pl
from jax.experimental.pallas import tpu as pltpu
from jax.experimental.pallas import tpu_sc as plsc

N_NODES = 100000
N_EDGES = 6400000
DIST_MIN = 1.2
DIST_MAX = 1.9
EPS = 0.1

NC = 2
NS = 16
L = 16
NW = NC * NS
PER_W = N_EDGES // NW
CHUNK = 4000
NCHUNKS = PER_W // CHUNK
GROUPS = CHUNK // L

_mesh = plsc.VectorSubcoreMesh(core_axis_name="c", subcore_axis_name="s")

_IDX = lambda: pltpu.VMEM((CHUNK,), jnp.int32)
_DAT = lambda: pltpu.VMEM((CHUNK,), jnp.float32)


@functools.partial(
    pl.kernel,
    mesh=_mesh,
    out_type=jax.ShapeDtypeStruct((NW, L), jnp.float32),
    scratch_types=[
        _IDX(), _IDX(),
        _IDX(), _IDX(),
        _IDX(), _IDX(),
        _DAT(), _DAT(), _DAT(),
        _DAT(), _DAT(), _DAT(),
        _DAT(), _DAT(), _DAT(),
        _DAT(), _DAT(), _DAT(),
        pltpu.VMEM((L,), jnp.float32),
        pltpu.VMEM_SHARED((N_NODES,), jnp.float32),
        pltpu.VMEM_SHARED((N_NODES,), jnp.float32),
        pltpu.VMEM_SHARED((N_NODES,), jnp.float32),
        pltpu.SemaphoreType.DMA,
        pltpu.SemaphoreType.DMA,
        pltpu.SemaphoreType.DMA,
    ],
)
def _drift_kernel(x0_hbm, x1_hbm, x2_hbm, ei_hbm, et_hbm, out_hbm,
                  si0, si1, di0, di1, et0, et1,
                  sx0, sy0, sz0, tx0, ty0, tz0,
                  sx1, sy1, sz1, tx1, ty1, tz1,
                  acc_v, x0_sh, x1_sh, x2_sh, sem_lin, sem_gat, sem_hgat):
    cid = lax.axis_index("c")
    sid = lax.axis_index("s")
    wid = sid * NC + cid
    base_w = wid * PER_W

    idx_set = ((si0, di0), (si1, di1))
    dat_set = ((et0, sx0, sy0, sz0, tx0, ty0, tz0),
               (et1, sx1, sy1, sz1, tx1, ty1, tz1))

    @pl.when(sid == 0)
    def _stage():
        pltpu.sync_copy(x0_hbm, x0_sh)
        pltpu.sync_copy(x1_hbm, x1_sh)
        pltpu.sync_copy(x2_hbm, x2_sh)

    plsc.subcore_barrier()

    def issue_lin(ci, p):
        base = base_w + ci * CHUNK
        si, di = idx_set[p]
        cps = [
            pltpu.make_async_copy(ei_hbm.at[pl.ds(base, CHUNK)], si, sem_lin),
            pltpu.make_async_copy(ei_hbm.at[pl.ds(N_EDGES + base, CHUNK)],
                                  di, sem_lin),
        ]
        for cp in cps:
            cp.start()
        return cps

    def issue_gat(ci, p):
        base = base_w + ci * CHUNK
        si, di = idx_set[p]
        et, sx, sy, sz, tx, ty, tz = dat_set[p]
        cps = [
            pltpu.make_async_copy(et_hbm.at[pl.ds(base, CHUNK)], et, sem_lin),
            pltpu.make_async_copy(x0_sh.at[si], sx, sem_gat),
            pltpu.make_async_copy(x1_sh.at[si], sy, sem_gat),
            pltpu.make_async_copy(x2_hbm.at[si], sz, sem_hgat),
            pltpu.make_async_copy(x0_sh.at[di], tx, sem_gat),
            pltpu.make_async_copy(x1_sh.at[di], ty, sem_gat),
            pltpu.make_async_copy(x2_sh.at[di], tz, sem_gat),
        ]
        for cp in cps:
            cp.start()
        return cps

    def wait_all(cps):
        for cp in cps:
            cp.wait()

    def compute(p, acc):
        et, sx, sy, sz, tx, ty, tz = dat_set[p]

        def group_body(gi, acc_in):
            o = pl.ds(gi * L, L)
            dx = sx[o] - tx[o]
            dy = sy[o] - ty[o]
            dz = sz[o] - tz[o]
            etv = et[o]
            d2 = jnp.maximum(dx * dx + dy * dy + dz * dz, 1e-30)

            bits = lax.bitcast_convert_type(d2, jnp.int32)
            seed = jnp.int32(0x5F3759DF) - lax.shift_right_arithmetic(bits, 1)
            y = lax.bitcast_convert_type(seed, jnp.float32)
            h = 0.5 * d2
            y = y * (1.5 - h * y * y)
            y = y * (1.5 - h * y * y)
            d = d2 * y

            drift = jnp.maximum(d - DIST_MAX,
                                jnp.maximum(DIST_MIN - d, 0.0))
            drift = jnp.where(etv == 0, 0.0, drift)
            return acc_in + drift

        return lax.fori_loop(0, GROUPS, group_body, acc, unroll=4)

    wait_all(issue_lin(0, 0))
    cps_l1 = issue_lin(1, 1)
    cps_g0 = issue_gat(0, 0)
    wait_all(cps_l1)
    wait_all(cps_g0)

    def pair_body(i, acc):
        ci = 2 * i
        cps_l = issue_lin(ci + 2, 0)
        cps_g = issue_gat(ci + 1, 1)
        acc = compute(0, acc)
        wait_all(cps_g)
        wait_all(cps_l)
        cps_l = issue_lin(ci + 3, 1)
        cps_g = issue_gat(ci + 2, 0)
        acc = compute(1, acc)
        wait_all(cps_g)
        wait_all(cps_l)
        return acc

    acc = lax.fori_loop(0, NCHUNKS // 2 - 1, pair_body,
                        jnp.zeros((L,), jnp.float32))
    cps_g = issue_gat(NCHUNKS - 1, 1)
    acc = compute(0, acc)
    wait_all(cps_g)
    acc = compute(1, acc)

    acc_v[...] = acc * EPS
    pltpu.sync_copy(acc_v, out_hbm.at[wid])


def kernel(x, e_type, e_index):
    x0 = x[:, 0]
    x1 = x[:, 1]
    x2 = x[:, 2]
    parts = _drift_kernel(x0, x1, x2,
                          e_index.astype(jnp.int32).reshape(-1),
                          e_type.astype(jnp.int32))
    return parts.sum()
